# baseline (speedup 1.0000x reference)
import functools
import math

import jax
import jax.numpy as jnp
from jax.experimental import pallas as pl
from jax.experimental.pallas import tpu as pltpu

_TEMP = 0.1
_BASE_TEMP = 0.1
_EPS = 1e-12
_INV_T = 10.0


_F8_SCALE = 64.0
_SIM_F8_SCALE = 16.0


def _norm_kernel(f_ref, out_ref):
    f = f_ref[...]
    r = jax.lax.rsqrt(jnp.maximum(jnp.sum(f * f, axis=1, keepdims=True), 1e-24))
    out_ref[...] = (f * (r * _F8_SCALE)).astype(jnp.float8_e4m3fn)


def _wsc_kernel(fi_ref, fj_ref, sw_ref, labj_ref, labi_ref, out_ref,
                s_acc, g_acc, c_acc, *, bi, bj, nj, cpad):
    i = pl.program_id(0)
    j = pl.program_id(1)

    @pl.when(j == 0)
    def _init():
        s_acc[...] = jnp.zeros_like(s_acc)
        g_acc[...] = jnp.zeros_like(g_acc)
        c_acc[...] = jnp.zeros_like(c_acc)

    sim = jax.lax.dot_general(fi_ref[...], fj_ref[...], (((1,), (1,)), ((), ())),
                              preferred_element_type=jnp.float32)
    sim = sim * (_INV_T / (_F8_SCALE * _F8_SCALE))

    labj = labj_ref[...]
    ohm = labj == jax.lax.broadcasted_iota(jnp.int32, (cpad, bj), 0)
    oh = ohm.astype(jnp.float8_e4m3fn)
    c_acc[...] += jnp.sum(ohm.astype(jnp.float32), axis=1, keepdims=True)

    jdiag = (i * bi) // bj

    @pl.when(jdiag != j)
    def _offdiag_block():
        s_acc[...] += jnp.sum(jnp.exp(sim - _INV_T), axis=1, keepdims=True)
        g_acc[...] += jax.lax.dot_general(
            (sim * _SIM_F8_SCALE).astype(jnp.float8_e4m3fn), oh,
            (((1,), (1,)), ((), ())), preferred_element_type=jnp.float32)

    @pl.when(jdiag == j)
    def _diag_block():
        offd = ((i * bi + jax.lax.broadcasted_iota(jnp.int32, (bi, bj), 0))
                != (j * bj + jax.lax.broadcasted_iota(jnp.int32, (bi, bj), 1)))
        s_acc[...] += jnp.sum(jnp.where(offd, jnp.exp(sim - _INV_T), 0.0),
                              axis=1, keepdims=True)
        simz = jnp.where(offd, sim * _SIM_F8_SCALE, 0.0)
        g_acc[...] += jax.lax.dot_general(
            simz.astype(jnp.float8_e4m3fn), oh, (((1,), (1,)), ((), ())),
            preferred_element_type=jnp.float32)

    @pl.when(j == nj - 1)
    def _emit():
        sw = sw_ref[...]
        ohi = (labi_ref[...] == jax.lax.broadcasted_iota(
            jnp.int32, (bi, cpad), 1)).astype(jnp.float32)
        sw_il = jnp.sum(sw * ohi, axis=1, keepdims=True)
        W = jnp.dot(sw, c_acc[...], preferred_element_type=jnp.float32) - sw_il
        P = jnp.sum(sw * g_acc[...], axis=1, keepdims=True) * (1.0 / _SIM_F8_SCALE)
        logden = _INV_T + jnp.log(s_acc[...] + _EPS)
        out_ref[...] = -(_TEMP / _BASE_TEMP) * (P - W * logden) / (W + _EPS)


@jax.jit
def kernel(features, labels, similarity_weights):
    B, D = features.shape
    C = similarity_weights.shape[1]
    cpad = 128
    bi, bj = 1024, 2048
    ni, nj = B // bi, B // bj

    lab32 = labels.astype(jnp.int32)
    labj2d = lab32.reshape(1, B)
    labi2d = lab32.reshape(B, 1)
    swp = jnp.zeros((B, cpad), jnp.float32).at[:, :C].set(similarity_weights)

    bn = 512
    fnorm = pl.pallas_call(
        _norm_kernel,
        grid=(B // bn,),
        in_specs=[pl.BlockSpec((bn, D), lambda n: (n, 0))],
        out_specs=pl.BlockSpec((bn, D), lambda n: (n, 0)),
        out_shape=jax.ShapeDtypeStruct((B, D), jnp.float8_e4m3fn),
        compiler_params=pltpu.CompilerParams(
            dimension_semantics=("arbitrary",)),
    )(features)

    out = pl.pallas_call(
        functools.partial(_wsc_kernel, bi=bi, bj=bj, nj=nj, cpad=cpad),
        grid=(ni, nj),
        in_specs=[
            pl.BlockSpec((bi, D), lambda i, j: (i, 0)),
            pl.BlockSpec((bj, D), lambda i, j: (j, 0)),
            pl.BlockSpec((bi, cpad), lambda i, j: (i, 0)),
            pl.BlockSpec((1, bj), lambda i, j: (0, j)),
            pl.BlockSpec((bi, 1), lambda i, j: (i, 0)),
        ],
        out_specs=pl.BlockSpec((bi, 1), lambda i, j: (i, 0)),
        out_shape=jax.ShapeDtypeStruct((B, 1), jnp.float32),
        scratch_shapes=[
            pltpu.VMEM((bi, 1), jnp.float32),
            pltpu.VMEM((bi, cpad), jnp.float32),
            pltpu.VMEM((cpad, 1), jnp.float32),
        ],
        compiler_params=pltpu.CompilerParams(
            dimension_semantics=("arbitrary", "arbitrary")),
    )(fnorm, fnorm, swp, labj2d, labi2d)
    return jnp.mean(out)


# speedup vs baseline: 1.0616x; 1.0616x over previous
import functools

import jax
import jax.numpy as jnp
from jax.experimental import pallas as pl
from jax.experimental.pallas import tpu as pltpu

_TEMP = 0.1
_BASE_TEMP = 0.1
_EPS = 1e-12
_INV_T = 10.0
_F8_SCALE = 64.0


def _norm_kernel(f_ref, out_ref):
    f = f_ref[...]
    r = jax.lax.rsqrt(jnp.maximum(jnp.sum(f * f, axis=1, keepdims=True), 1e-24))
    out_ref[...] = (f * (r * _F8_SCALE)).astype(jnp.float8_e4m3fn)


def _wsc_kernel(fi_ref, fj_ref, labi_ref, labj_ref, sw_ref, labcol_ref,
                out_ref, sr_acc, sc_acc, g_acc, c_acc, *, b, ni, nt, cpad):
    t = pl.program_id(0)
    i = t % ni
    off = t // ni
    j = (i + off) % ni

    @pl.when(t == 0)
    def _init():
        sr_acc[...] = jnp.zeros_like(sr_acc)
        sc_acc[...] = jnp.zeros_like(sc_acc)
        g_acc[...] = jnp.zeros_like(g_acc)
        c_acc[...] = jnp.zeros_like(c_acc)

    sim = jax.lax.dot_general(fi_ref[...], fj_ref[...], (((1,), (1,)), ((), ())),
                              preferred_element_type=jnp.float32)
    sim = sim * (_INV_T / (_F8_SCALE * _F8_SCALE))

    labj = labj_ref[...]
    ohj = (labj == jax.lax.broadcasted_iota(jnp.int32, (cpad, b), 0)
           ).astype(jnp.bfloat16)

    @pl.when(off == 0)
    def _diag_block():
        base = i * b
        offd = (jax.lax.broadcasted_iota(jnp.int32, (b, b), 0)
                != jax.lax.broadcasted_iota(jnp.int32, (b, b), 1))
        e = jnp.where(offd, jnp.exp(sim - _INV_T), 0.0)
        sr_acc[pl.ds(base, b), :] += jnp.sum(e, axis=1, keepdims=True)
        simz = jnp.where(offd, sim, 0.0)
        g_acc[pl.ds(base, b), :] += jax.lax.dot_general(
            simz.astype(jnp.bfloat16), ohj, (((1,), (1,)), ((), ())),
            preferred_element_type=jnp.float32)
        c_acc[...] += jnp.sum(ohj.astype(jnp.float32), axis=1, keepdims=True)

    @pl.when(off != 0)
    def _offdiag_block():
        e = jnp.exp(sim - _INV_T)
        sr_acc[pl.ds(i * b, b), :] += jnp.sum(e, axis=1, keepdims=True)
        g_acc[pl.ds(i * b, b), :] += jax.lax.dot_general(
            sim.astype(jnp.bfloat16), ohj, (((1,), (1,)), ((), ())),
            preferred_element_type=jnp.float32)

        @pl.when(off < ni // 2)
        def _col_side():
            sc_acc[:, pl.ds(j * b, b)] += jnp.sum(e, axis=0, keepdims=True)
            labi = labi_ref[...]
            ohi = (labi == jax.lax.broadcasted_iota(jnp.int32, (cpad, b), 0)
                   ).astype(jnp.bfloat16)
            g_acc[pl.ds(j * b, b), :] += jax.lax.dot_general(
                sim.astype(jnp.bfloat16), ohi, (((0,), (1,)), ((), ())),
                preferred_element_type=jnp.float32)

    @pl.when(t == nt - 1)
    def _emit():
        B = ni * b
        S = sr_acc[...] + jnp.transpose(sc_acc[...])
        sw = sw_ref[...]
        ohi = (labcol_ref[...] == jax.lax.broadcasted_iota(
            jnp.int32, (B, cpad), 1)).astype(jnp.float32)
        sw_il = jnp.sum(sw * ohi, axis=1, keepdims=True)
        W = jnp.dot(sw, c_acc[...], preferred_element_type=jnp.float32) - sw_il
        P = jnp.sum(sw * g_acc[...], axis=1, keepdims=True)
        logden = _INV_T + jnp.log(S + _EPS)
        out_ref[...] = -(_TEMP / _BASE_TEMP) * (P - W * logden) / (W + _EPS)


@jax.jit
def kernel(features, labels, similarity_weights):
    B, D = features.shape
    C = similarity_weights.shape[1]
    cpad = 128
    b = 1024
    ni = B // b
    nt = ni * (ni // 2 + 1)

    lab32 = labels.astype(jnp.int32)
    labrow = lab32.reshape(1, B)
    labcol = lab32.reshape(B, 1)
    swp = jnp.zeros((B, cpad), jnp.float32).at[:, :C].set(similarity_weights)

    bn = 1024
    fnorm = pl.pallas_call(
        _norm_kernel,
        grid=(B // bn,),
        in_specs=[pl.BlockSpec((bn, D), lambda n: (n, 0))],
        out_specs=pl.BlockSpec((bn, D), lambda n: (n, 0)),
        out_shape=jax.ShapeDtypeStruct((B, D), jnp.float8_e4m3fn),
        compiler_params=pltpu.CompilerParams(
            dimension_semantics=("arbitrary",)),
    )(features)

    out = pl.pallas_call(
        functools.partial(_wsc_kernel, b=b, ni=ni, nt=nt, cpad=cpad),
        grid=(nt,),
        in_specs=[
            pl.BlockSpec((b, D), lambda t: (t % ni, 0)),
            pl.BlockSpec((b, D), lambda t: ((t % ni + t // ni) % ni, 0)),
            pl.BlockSpec((1, b), lambda t: (0, t % ni)),
            pl.BlockSpec((1, b), lambda t: (0, (t % ni + t // ni) % ni)),
            pl.BlockSpec((B, cpad), lambda t: (0, 0)),
            pl.BlockSpec((B, 1), lambda t: (0, 0)),
        ],
        out_specs=pl.BlockSpec((B, 1), lambda t: (0, 0)),
        out_shape=jax.ShapeDtypeStruct((B, 1), jnp.float32),
        scratch_shapes=[
            pltpu.VMEM((B, 1), jnp.float32),
            pltpu.VMEM((1, B), jnp.float32),
            pltpu.VMEM((B, cpad), jnp.float32),
            pltpu.VMEM((cpad, 1), jnp.float32),
        ],
        compiler_params=pltpu.CompilerParams(
            dimension_semantics=("arbitrary",)),
    )(fnorm, fnorm, labrow, labrow, swp, labcol)
    return jnp.mean(out)


# speedup vs baseline: 1.1301x; 1.0645x over previous
import functools

import jax
import jax.numpy as jnp
from jax.experimental import pallas as pl
from jax.experimental.pallas import tpu as pltpu

_TEMP = 0.1
_BASE_TEMP = 0.1
_EPS = 1e-12
_INV_T = 10.0
_F8_SCALE = 64.0


def _wsc_kernel(f_ref, labi_ref, labj_ref, sw_ref, labcol_ref, out_ref,
                fn8, sr_acc, sc_acc, g_acc, c_acc, *, b, ni, na, bn, nt, cpad):
    t = pl.program_id(0)

    @pl.when(t == 0)
    def _init():
        sr_acc[...] = jnp.zeros_like(sr_acc)
        sc_acc[...] = jnp.zeros_like(sc_acc)
        g_acc[...] = jnp.zeros_like(g_acc)
        c_acc[...] = jnp.zeros_like(c_acc)

    @pl.when(t < na)
    def _normalize():
        f = f_ref[...]
        r = jax.lax.rsqrt(jnp.maximum(jnp.sum(f * f, axis=1, keepdims=True),
                                      1e-24))
        fn8[pl.ds(t * bn, bn), :] = (f * (r * _F8_SCALE)).astype(
            jnp.float8_e4m3fn)

    @pl.when(t >= na)
    def _main():
        tc = t - na
        i = tc % ni
        off = tc // ni
        j = (i + off) % ni

        fi = fn8[pl.ds(i * b, b), :]
        fj = fn8[pl.ds(j * b, b), :]
        sim = jax.lax.dot_general(fi, fj, (((1,), (1,)), ((), ())),
                                  preferred_element_type=jnp.float32)
        sim = sim * (_INV_T / (_F8_SCALE * _F8_SCALE))

        labj = labj_ref[...]
        ohj = (labj == jax.lax.broadcasted_iota(jnp.int32, (cpad, b), 0)
               ).astype(jnp.bfloat16)

        @pl.when(off == 0)
        def _diag_block():
            offd = (jax.lax.broadcasted_iota(jnp.int32, (b, b), 0)
                    != jax.lax.broadcasted_iota(jnp.int32, (b, b), 1))
            e = jnp.where(offd, jnp.exp(sim - _INV_T), 0.0)
            sr_acc[pl.ds(i * b, b), :] += jnp.sum(e, axis=1, keepdims=True)
            simz = jnp.where(offd, sim, 0.0)
            g_acc[pl.ds(i * b, b), :] += jax.lax.dot_general(
                simz.astype(jnp.bfloat16), ohj, (((1,), (1,)), ((), ())),
                preferred_element_type=jnp.float32)
            c_acc[...] += jnp.sum(ohj.astype(jnp.float32), axis=1,
                                  keepdims=True)

        @pl.when(off != 0)
        def _offdiag_block():
            e = jnp.exp(sim - _INV_T)
            sr_acc[pl.ds(i * b, b), :] += jnp.sum(e, axis=1, keepdims=True)
            g_acc[pl.ds(i * b, b), :] += jax.lax.dot_general(
                sim.astype(jnp.bfloat16), ohj, (((1,), (1,)), ((), ())),
                preferred_element_type=jnp.float32)

            @pl.when(off < ni // 2)
            def _col_side():
                sc_acc[:, pl.ds(j * b, b)] += jnp.sum(e, axis=0, keepdims=True)
                labi = labi_ref[...]
                ohi = (labi == jax.lax.broadcasted_iota(jnp.int32, (cpad, b), 0)
                       ).astype(jnp.bfloat16)
                g_acc[pl.ds(j * b, b), :] += jax.lax.dot_general(
                    sim.astype(jnp.bfloat16), ohi, (((0,), (1,)), ((), ())),
                    preferred_element_type=jnp.float32)

    @pl.when(t == na + nt - 1)
    def _emit():
        B = ni * b
        S = sr_acc[...] + jnp.transpose(sc_acc[...])
        sw = sw_ref[...]
        ohi = (labcol_ref[...] == jax.lax.broadcasted_iota(
            jnp.int32, (B, cpad), 1)).astype(jnp.float32)
        sw_il = jnp.sum(sw * ohi, axis=1, keepdims=True)
        W = jnp.dot(sw, c_acc[...], preferred_element_type=jnp.float32) - sw_il
        P = jnp.sum(sw * g_acc[...], axis=1, keepdims=True)
        logden = _INV_T + jnp.log(S + _EPS)
        out_ref[...] = -(_TEMP / _BASE_TEMP) * (P - W * logden) / (W + _EPS)


@jax.jit
def kernel(features, labels, similarity_weights):
    B, D = features.shape
    C = similarity_weights.shape[1]
    cpad = 128
    b = 1024
    bn = 1024
    ni = B // b
    na = B // bn
    nt = ni * (ni // 2 + 1)

    lab32 = labels.astype(jnp.int32)
    labrow = lab32.reshape(1, B)
    labcol = lab32.reshape(B, 1)
    swp = jnp.zeros((B, cpad), jnp.float32).at[:, :C].set(similarity_weights)

    def _i_map(t):
        tc = jnp.maximum(t - na, 0)
        return (0, tc % ni)

    def _j_map(t):
        tc = jnp.maximum(t - na, 0)
        return (0, (tc % ni + tc // ni) % ni)

    out = pl.pallas_call(
        functools.partial(_wsc_kernel, b=b, ni=ni, na=na, bn=bn, nt=nt,
                          cpad=cpad),
        grid=(na + nt,),
        in_specs=[
            pl.BlockSpec((bn, D), lambda t: (jnp.minimum(t, na - 1), 0)),
            pl.BlockSpec((1, b), _i_map),
            pl.BlockSpec((1, b), _j_map),
            pl.BlockSpec((B, cpad), lambda t: (0, 0)),
            pl.BlockSpec((B, 1), lambda t: (0, 0)),
        ],
        out_specs=pl.BlockSpec((B, 1), lambda t: (0, 0)),
        out_shape=jax.ShapeDtypeStruct((B, 1), jnp.float32),
        scratch_shapes=[
            pltpu.VMEM((B, D), jnp.float8_e4m3fn),
            pltpu.VMEM((B, 1), jnp.float32),
            pltpu.VMEM((1, B), jnp.float32),
            pltpu.VMEM((B, cpad), jnp.float32),
            pltpu.VMEM((cpad, 1), jnp.float32),
        ],
        compiler_params=pltpu.CompilerParams(
            dimension_semantics=("arbitrary",)),
    )(features, labrow, labrow, swp, labcol)
    return jnp.mean(out)
